# baseline (speedup 1.0000x reference)
import functools

import jax
import jax.numpy as jnp
from jax import lax
from jax.experimental import pallas as pl

B, T, DIM = 4, 100000, 3
HID = 32
CDIM = 32
RESO = 128
NBLK = 5
PAD = 0.1
N = B * T
R2 = RESO * RESO


_K1_BLK = 16000
_K1_GRID = N // _K1_BLK


def _k1_body(p_ref, wpos_ref, bpos_ref, w0_ref, b0_ref, w1_ref, b1_ref,
             ws_ref, net_ref, idx_ref):
    i = pl.program_id(0)
    pblk = p_ref[...]
    x0 = pblk[0:1, :]
    z0 = pblk[2:3, :]
    c = jnp.float32(1 + PAD + 10e-4)
    xs = jnp.clip(x0 / c + 0.5, 0.0, 1 - 10e-6)
    zs = jnp.clip(z0 / c + 0.5, 0.0, 1 - 10e-6)
    xi = (xs * RESO).astype(jnp.int32)
    zi = (zs * RESO).astype(jnp.int32)
    n_glob = lax.broadcasted_iota(jnp.int32, (1, _K1_BLK), 1) + i * _K1_BLK
    boff = ((n_glob >= T).astype(jnp.int32) + (n_glob >= 2 * T).astype(jnp.int32)
            + (n_glob >= 3 * T).astype(jnp.int32))
    idx_ref[0] = xi + RESO * zi + R2 * boff

    net = lax.dot_general(pblk, wpos_ref[...], (((0,), (0,)), ((), ())),
                          preferred_element_type=jnp.float32) + bpos_ref[...]
    h = jnp.maximum(net, 0.0)
    n1 = jnp.dot(h, w0_ref[...], preferred_element_type=jnp.float32) + b0_ref[...]
    d = jnp.dot(jnp.maximum(n1, 0.0), w1_ref[...],
                preferred_element_type=jnp.float32) + b1_ref[...]
    net_ref[...] = jnp.dot(net, ws_ref[...], preferred_element_type=jnp.float32) + d


def _k1(p_fm, wpos, bpos, w0, b0, w1, b1, ws):
    wspec = pl.BlockSpec((None,), lambda i: (0,))
    net0, idx3 = pl.pallas_call(
        _k1_body,
        grid=(_K1_GRID,),
        in_specs=[
            pl.BlockSpec((3, _K1_BLK), lambda i: (0, i)),
            pl.BlockSpec((DIM, 2 * HID), lambda i: (0, 0)),
            pl.BlockSpec((1, 2 * HID), lambda i: (0, 0)),
            pl.BlockSpec((2 * HID, HID), lambda i: (0, 0)),
            pl.BlockSpec((1, HID), lambda i: (0, 0)),
            pl.BlockSpec((HID, HID), lambda i: (0, 0)),
            pl.BlockSpec((1, HID), lambda i: (0, 0)),
            pl.BlockSpec((2 * HID, HID), lambda i: (0, 0)),
        ],
        out_specs=[
            pl.BlockSpec((_K1_BLK, HID), lambda i: (i, 0)),
            pl.BlockSpec((1, 1, _K1_BLK), lambda i: (i, 0, 0)),
        ],
        out_shape=[
            jax.ShapeDtypeStruct((N, HID), jnp.float32),
            jax.ShapeDtypeStruct((_K1_GRID, 1, _K1_BLK), jnp.int32),
        ],
    )(p_fm, wpos, bpos, w0, b0, w1, b1, ws)
    return net0, idx3.reshape(N)


_KB_BLK = 8000
_KB_GRID = N // _KB_BLK


def _kb_body(with_c, xa_ref, xb_ref, w0a_ref, w0b_ref, b0_ref, w1_ref, b1_ref,
             wsa_ref, wsb_ref, wc_ref, bc_ref, out_ref):
    xa = xa_ref[...]
    xb = xb_ref[...]
    f32 = jnp.float32
    n1 = (jnp.dot(jnp.maximum(xa, 0.0), w0a_ref[...], preferred_element_type=f32)
          + jnp.dot(jnp.maximum(xb, 0.0), w0b_ref[...], preferred_element_type=f32)
          + b0_ref[...])
    d = jnp.dot(jnp.maximum(n1, 0.0), w1_ref[...], preferred_element_type=f32) + b1_ref[...]
    out = (jnp.dot(xa, wsa_ref[...], preferred_element_type=f32)
           + jnp.dot(xb, wsb_ref[...], preferred_element_type=f32) + d)
    if with_c:
        out = jnp.dot(out, wc_ref[...], preferred_element_type=f32) + bc_ref[...]
    out_ref[...] = out


def _kblock(xa, xb, w0a, w0b, b0, w1, b1, wsa, wsb, wc, bc, with_c):
    return pl.pallas_call(
        functools.partial(_kb_body, with_c),
        grid=(_KB_GRID,),
        in_specs=[
            pl.BlockSpec((_KB_BLK, HID), lambda i: (i, 0)),
            pl.BlockSpec((_KB_BLK, HID), lambda i: (i, 0)),
            pl.BlockSpec((HID, HID), lambda i: (0, 0)),
            pl.BlockSpec((HID, HID), lambda i: (0, 0)),
            pl.BlockSpec((1, HID), lambda i: (0, 0)),
            pl.BlockSpec((HID, HID), lambda i: (0, 0)),
            pl.BlockSpec((1, HID), lambda i: (0, 0)),
            pl.BlockSpec((HID, HID), lambda i: (0, 0)),
            pl.BlockSpec((HID, HID), lambda i: (0, 0)),
            pl.BlockSpec((HID, CDIM), lambda i: (0, 0)),
            pl.BlockSpec((1, CDIM), lambda i: (0, 0)),
        ],
        out_specs=pl.BlockSpec((_KB_BLK, CDIM), lambda i: (i, 0)),
        out_shape=jax.ShapeDtypeStruct((N, CDIM), jnp.float32),
    )(xa, xb, w0a, w0b, b0, w1, b1, wsa, wsb, wc, bc)


def kernel(p, fc_pos_w, fc_pos_b, blk_fc0_w, blk_fc0_b, blk_fc1_w, blk_fc1_b,
           blk_sc_w, fc_c_w, fc_c_b):
    p_fm = p.reshape(N, DIM).T
    net, idx_flat = _k1(p_fm, fc_pos_w, fc_pos_b.reshape(1, -1),
                        blk_fc0_w[0], blk_fc0_b[0].reshape(1, -1),
                        blk_fc1_w[0], blk_fc1_b[0].reshape(1, -1),
                        blk_sc_w[0])

    nseg = B * R2
    cnt = jax.ops.segment_sum(jnp.ones((N,), jnp.float32), idx_flat,
                              num_segments=nseg)

    for i in range(1, NBLK):
        seg = jax.ops.segment_max(net, idx_flat, num_segments=nseg)
        seg = jnp.where(cnt[:, None] > 0, seg, 0.0)
        pooled = seg[idx_flat]
        with_c = i == NBLK - 1
        net = _kblock(net, pooled,
                      blk_fc0_w[i, :HID], blk_fc0_w[i, HID:],
                      blk_fc0_b[i].reshape(1, -1),
                      blk_fc1_w[i], blk_fc1_b[i].reshape(1, -1),
                      blk_sc_w[i, :HID], blk_sc_w[i, HID:],
                      fc_c_w, fc_c_b.reshape(1, -1), with_c)

    csum = jax.ops.segment_sum(net, idx_flat, num_segments=nseg)
    cmean = csum / jnp.clip(cnt[:, None], 1.0, None)
    return cmean.reshape(B, R2, CDIM).transpose(0, 2, 1).reshape(B, CDIM, RESO, RESO)
